# Initial kernel scaffold; baseline (speedup 1.0000x reference)
import jax
import jax.numpy as jnp
from jax.experimental import pallas as pl


def kernel(feats, batch):
    raise NotImplementedError("write your pallas kernel here")


# speedup vs baseline: 2.2807x; 2.2807x over previous
import jax
import jax.numpy as jnp
from jax import lax
from jax.experimental import pallas as pl
from jax.experimental.pallas import tpu as pltpu
from jax.experimental.pallas import tpu_sc as plsc

N_ROWS = 160000
N_FEATS = 256
N_SEG = 10000

NUM_CORES = 2
NUM_SUBCORES = 16
LANES = 16

N_PASS = 2
DQ = N_FEATS // (NUM_CORES * N_PASS)
ROWS_PER_TILE = N_ROWS // NUM_SUBCORES
CHUNK = 400
SUB = 80
N_CHUNK = ROWS_PER_TILE // CHUNK
N_SUB = CHUNK // SUB

SEG_PER_TILE = 640
S_PAD = SEG_PER_TILE * NUM_SUBCORES
SEG_CHUNK = 80
FULL_SEG_CHUNKS = SEG_PER_TILE // SEG_CHUNK
LAST_SEG_CHUNKS = (N_SEG - 15 * SEG_PER_TILE) // SEG_CHUNK


def _body(feats_hbm, batch_hbm, out_hbm, fbuf, ibuf, ones, sbuf, cbuf,
          sums_sh, cnt_sh):
  core = lax.axis_index("c")
  tile = lax.axis_index("s")
  row_base = tile * ROWS_PER_TILE
  seg_base = tile * SEG_PER_TILE

  zeros16 = jnp.zeros((LANES,), jnp.float32)
  ones16 = jnp.ones((LANES,), jnp.float32)

  def zero_sbuf(s, _):
    for v in range(DQ // LANES):
      sbuf[s, pl.ds(v * LANES, LANES)] = zeros16
    return _
  lax.fori_loop(0, SEG_CHUNK, zero_sbuf, 0)

  def zero_cbuf(s, _):
    cbuf[s, :] = zeros16
    return _
  lax.fori_loop(0, SEG_PER_TILE, zero_cbuf, 0)

  def fill_ones(s, _):
    ones[s, :] = ones16
    return _
  lax.fori_loop(0, SUB, fill_ones, 0)

  pltpu.sync_copy(cbuf, cnt_sh.at[pl.ds(pl.multiple_of(seg_base, 8),
                                        SEG_PER_TILE)])

  n_chunks = jnp.where(tile == NUM_SUBCORES - 1, LAST_SEG_CHUNKS,
                       FULL_SEG_CHUNKS)

  for p in range(N_PASS):
    col0 = core * (N_PASS * DQ) + p * DQ

    for k in range(FULL_SEG_CHUNKS):
      base = pl.multiple_of(seg_base + k * SEG_CHUNK, 8)
      pltpu.sync_copy(sbuf, sums_sh.at[pl.ds(base, SEG_CHUNK)])

    plsc.subcore_barrier()

    def accum(k, _):
      row0 = pl.multiple_of(row_base + k * CHUNK, 8)
      pltpu.sync_copy(
          feats_hbm.at[pl.ds(row0, CHUNK), pl.ds(col0, DQ)], fbuf)
      for j in range(N_SUB):
        pltpu.sync_copy(
            batch_hbm.at[pl.ds(pl.multiple_of(row0 + j * SUB, 8), SUB)],
            ibuf.at[j])
      for j in range(N_SUB):
        pltpu.sync_copy(fbuf.at[pl.ds(j * SUB, SUB)],
                        sums_sh.at[ibuf.at[j]], add=True)
        if p == 0:
          pltpu.sync_copy(ones, cnt_sh.at[ibuf.at[j]], add=True)
      return _
    lax.fori_loop(0, N_CHUNK, accum, 0)

    plsc.subcore_barrier()

    if p == 0:
      pltpu.sync_copy(cnt_sh.at[pl.ds(pl.multiple_of(seg_base, 8),
                                      SEG_PER_TILE)], cbuf)

    def finalize(k, _):
      base = pl.multiple_of(seg_base + k * SEG_CHUNK, 8)
      pltpu.sync_copy(sums_sh.at[pl.ds(base, SEG_CHUNK)], sbuf)

      def div_one(s, _2):
        cntv = cbuf[k * SEG_CHUNK + s, :]
        inv = (ones16 / jnp.maximum(cntv, ones16))[0]
        for v in range(DQ // LANES):
          sl = pl.ds(v * LANES, LANES)
          sbuf[s, sl] = sbuf[s, sl] * inv
        return _2
      lax.fori_loop(0, SEG_CHUNK, div_one, 0)

      pltpu.sync_copy(sbuf, out_hbm.at[pl.ds(base, SEG_CHUNK),
                                       pl.ds(col0, DQ)])
      return _
    lax.fori_loop(0, n_chunks, finalize, 0)

    if p + 1 < N_PASS:
      def rezero_sbuf(s, _):
        for v in range(DQ // LANES):
          sbuf[s, pl.ds(v * LANES, LANES)] = zeros16
        return _
      lax.fori_loop(0, SEG_CHUNK, rezero_sbuf, 0)
      plsc.subcore_barrier()


@jax.jit
def _pool_mean(feats, batch):
  mesh = plsc.VectorSubcoreMesh(core_axis_name="c", subcore_axis_name="s")
  return pl.kernel(
      _body,
      out_type=jax.ShapeDtypeStruct((N_SEG, N_FEATS), jnp.float32),
      mesh=mesh,
      compiler_params=pltpu.CompilerParams(use_tc_tiling_on_sc=False),
      scratch_types=[
          pltpu.VMEM((CHUNK, DQ), jnp.float32),
          pltpu.VMEM((N_SUB, SUB), jnp.int32),
          pltpu.VMEM((SUB, LANES), jnp.float32),
          pltpu.VMEM((SEG_CHUNK, DQ), jnp.float32),
          pltpu.VMEM((SEG_PER_TILE, LANES), jnp.float32),
          pltpu.VMEM_SHARED((S_PAD, DQ), jnp.float32),
          pltpu.VMEM_SHARED((S_PAD, LANES), jnp.float32),
      ],
  )(feats, batch)


def kernel(feats, batch):
  return _pool_mean(feats, batch.astype(jnp.int32))


# speedup vs baseline: 3.4271x; 1.5026x over previous
import jax
import jax.numpy as jnp
from jax import lax
from jax.experimental import pallas as pl
from jax.experimental.pallas import tpu as pltpu
from jax.experimental.pallas import tpu_sc as plsc

N_ROWS = 160000
N_FEATS = 256
N_SEG = 10000

NUM_CORES = 2
NUM_SUBCORES = 16
LANES = 16

N_PASS = 2
DQ = N_FEATS // (NUM_CORES * N_PASS)
ROWS_PER_TILE = N_ROWS // NUM_SUBCORES
CHUNK = 400
SUB = 80
N_CHUNK = ROWS_PER_TILE // CHUNK
N_SUB = CHUNK // SUB
IDX_ROWS = ROWS_PER_TILE // SUB

SEG_PER_TILE = 640
S_PAD = SEG_PER_TILE * NUM_SUBCORES
SEG_CHUNK = 80
FULL_SEG_CHUNKS = SEG_PER_TILE // SEG_CHUNK
LAST_SEG_CHUNKS = (N_SEG - 15 * SEG_PER_TILE) // SEG_CHUNK


def _body(feats_hbm, batch_hbm, out_hbm, fbuf, ibuf, ones, sbuf, cbuf,
          sums_sh, cnt_sh, lsem, ssem, csem):
  core = lax.axis_index("c")
  tile = lax.axis_index("s")
  row_base = tile * ROWS_PER_TILE
  seg_base = tile * SEG_PER_TILE

  zeros16 = jnp.zeros((LANES,), jnp.float32)
  ones16 = jnp.ones((LANES,), jnp.float32)

  def zero_sbuf(s, _):
    for v in range(DQ // LANES):
      sbuf[s, pl.ds(v * LANES, LANES)] = zeros16
    return _
  lax.fori_loop(0, SEG_CHUNK, zero_sbuf, 0)

  def zero_cbuf(s, _):
    cbuf[s, :] = zeros16
    return _
  lax.fori_loop(0, SEG_PER_TILE, zero_cbuf, 0)

  def fill_ones(s, _):
    ones[s, :] = ones16
    return _
  lax.fori_loop(0, SUB, fill_ones, 0)

  pltpu.sync_copy(cbuf, cnt_sh.at[pl.ds(pl.multiple_of(seg_base, 8),
                                        SEG_PER_TILE)])
  pltpu.sync_copy(batch_hbm.at[pl.ds(tile * IDX_ROWS, IDX_ROWS)], ibuf)

  n_chunks = jnp.where(tile == NUM_SUBCORES - 1, LAST_SEG_CHUNKS,
                       FULL_SEG_CHUNKS)

  def start_load(k, slot, col0):
    row0 = pl.multiple_of(row_base + k * CHUNK, 8)
    return pltpu.async_copy(
        feats_hbm.at[pl.ds(row0, CHUNK), pl.ds(col0, DQ)], fbuf.at[slot],
        lsem)

  for p in range(N_PASS):
    col0 = core * (N_PASS * DQ) + p * DQ

    for k in range(FULL_SEG_CHUNKS):
      base = pl.multiple_of(seg_base + k * SEG_CHUNK, 8)
      pltpu.sync_copy(sbuf, sums_sh.at[pl.ds(base, SEG_CHUNK)])

    plsc.subcore_barrier()

    loads = {}
    scats = {}
    loads[0] = start_load(0, 0, col0)
    for k in range(N_CHUNK):
      loads[k].wait()
      if k >= 1:
        for d in scats.pop(k - 1):
          d.wait()
      if k + 1 < N_CHUNK:
        loads[k + 1] = start_load(k + 1, (k + 1) % 2, col0)
      batch_descs = []
      for j in range(N_SUB):
        batch_descs.append(pltpu.async_copy(
            fbuf.at[k % 2].at[pl.ds(j * SUB, SUB)],
            sums_sh.at[ibuf.at[k * N_SUB + j]], ssem, add=True))
        if p == 0:
          batch_descs.append(pltpu.async_copy(
              ones, cnt_sh.at[ibuf.at[k * N_SUB + j]], csem, add=True))
      scats[k] = batch_descs
    for d in scats.pop(N_CHUNK - 1):
      d.wait()

    plsc.subcore_barrier()

    if p == 0:
      pltpu.sync_copy(cnt_sh.at[pl.ds(pl.multiple_of(seg_base, 8),
                                      SEG_PER_TILE)], cbuf)

    def finalize(k, _):
      base = pl.multiple_of(seg_base + k * SEG_CHUNK, 8)
      pltpu.sync_copy(sums_sh.at[pl.ds(base, SEG_CHUNK)], sbuf)

      def div_one(s, _2):
        cntv = cbuf[k * SEG_CHUNK + s, :]
        inv = (ones16 / jnp.maximum(cntv, ones16))[0]
        for v in range(DQ // LANES):
          sl = pl.ds(v * LANES, LANES)
          sbuf[s, sl] = sbuf[s, sl] * inv
        return _2
      lax.fori_loop(0, SEG_CHUNK, div_one, 0)

      pltpu.sync_copy(sbuf, out_hbm.at[pl.ds(base, SEG_CHUNK),
                                       pl.ds(col0, DQ)])
      return _
    lax.fori_loop(0, n_chunks, finalize, 0)

    if p + 1 < N_PASS:
      def rezero_sbuf(s, _):
        for v in range(DQ // LANES):
          sbuf[s, pl.ds(v * LANES, LANES)] = zeros16
        return _
      lax.fori_loop(0, SEG_CHUNK, rezero_sbuf, 0)
      plsc.subcore_barrier()


@jax.jit
def _pool_mean(feats, batch2d):
  mesh = plsc.VectorSubcoreMesh(core_axis_name="c", subcore_axis_name="s")
  return pl.kernel(
      _body,
      out_type=jax.ShapeDtypeStruct((N_SEG, N_FEATS), jnp.float32),
      mesh=mesh,
      compiler_params=pltpu.CompilerParams(use_tc_tiling_on_sc=False),
      scratch_types=[
          pltpu.VMEM((2, CHUNK, DQ), jnp.float32),
          pltpu.VMEM((IDX_ROWS, SUB), jnp.int32),
          pltpu.VMEM((SUB, LANES), jnp.float32),
          pltpu.VMEM((SEG_CHUNK, DQ), jnp.float32),
          pltpu.VMEM((SEG_PER_TILE, LANES), jnp.float32),
          pltpu.VMEM_SHARED((S_PAD, DQ), jnp.float32),
          pltpu.VMEM_SHARED((S_PAD, LANES), jnp.float32),
          pltpu.SemaphoreType.DMA,
          pltpu.SemaphoreType.DMA,
          pltpu.SemaphoreType.DMA,
      ],
  )(feats, batch2d)


def kernel(feats, batch):
  batch2d = batch.astype(jnp.int32).reshape(N_ROWS // SUB, SUB)
  return _pool_mean(feats, batch2d)


# speedup vs baseline: 4.9889x; 1.4557x over previous
import jax
import jax.numpy as jnp
from jax import lax
from jax.experimental import pallas as pl
from jax.experimental.pallas import tpu as pltpu
from jax.experimental.pallas import tpu_sc as plsc

N_ROWS = 160000
N_FEATS = 256
N_SEG = 10000

NUM_CORES = 2
NUM_SUBCORES = 16
LANES = 16

DHALF = N_FEATS // NUM_CORES
ROWS_PER_TILE = N_ROWS // NUM_SUBCORES
SUB = 80
N_CHUNK = ROWS_PER_TILE // SUB
NRING = 2

SEG_PER_TILE = 640
S_PAD = SEG_PER_TILE * NUM_SUBCORES
SEG_CHUNK = 40
FULL_SEG_CHUNKS = SEG_PER_TILE // SEG_CHUNK
LAST_SEG_CHUNKS = (N_SEG - 15 * SEG_PER_TILE) // SEG_CHUNK

CNT_BATCH = 25


def _counts_body(batch_hbm, cnt_hbm, ibuf, ones, zbuf, cnt_sh, csem):
  core = lax.axis_index("c")
  tile = lax.axis_index("s")
  seg_base = tile * SEG_PER_TILE

  zeros16 = jnp.zeros((LANES,), jnp.float32)
  ones16 = jnp.ones((LANES,), jnp.float32)

  def init_rows(s, _):
    zbuf[s, :] = zeros16
    ones[s, :] = ones16
    return _
  lax.fori_loop(0, SUB, init_rows, 0)

  for k in range(SEG_PER_TILE // SUB):
    pltpu.sync_copy(zbuf, cnt_sh.at[pl.ds(seg_base + k * SUB, SUB)])

  plsc.subcore_barrier()

  pltpu.sync_copy(batch_hbm.at[tile], ibuf)
  groups = {}
  for g in range(N_CHUNK // CNT_BATCH):
    if g >= 1:
      for d in groups.pop(g - 1):
        d.wait()
    descs = []
    for j in range(CNT_BATCH):
      descs.append(pltpu.async_copy(
          ones, cnt_sh.at[ibuf.at[g * CNT_BATCH + j]], csem, add=True))
    groups[g] = descs
  for g in sorted(groups):
    for d in groups.pop(g):
      d.wait()

  plsc.subcore_barrier()

  @pl.when(core == 0)
  def _():
    pltpu.sync_copy(cnt_sh.at[pl.ds(seg_base, SEG_PER_TILE)],
                    cnt_hbm.at[pl.ds(seg_base, SEG_PER_TILE)])


def _pool_body(feats_hbm, batch_hbm, cnt_hbm, out_hbm, fbuf, ibuf, sbuf,
               cbuf, sums_sh, lsem):
  core = lax.axis_index("c")
  tile = lax.axis_index("s")
  col0 = core * DHALF
  row_base = tile * ROWS_PER_TILE
  seg_base = tile * SEG_PER_TILE

  zeros16 = jnp.zeros((LANES,), jnp.float32)
  ones16 = jnp.ones((LANES,), jnp.float32)

  def zero_sbuf(s, _):
    for v in range(DHALF // LANES):
      sbuf[s, pl.ds(v * LANES, LANES)] = zeros16
    return _
  lax.fori_loop(0, SEG_CHUNK, zero_sbuf, 0)

  for k in range(FULL_SEG_CHUNKS):
    base = pl.multiple_of(seg_base + k * SEG_CHUNK, 8)
    pltpu.sync_copy(sbuf, sums_sh.at[pl.ds(base, SEG_CHUNK)])

  pltpu.sync_copy(batch_hbm.at[tile], ibuf)

  plsc.subcore_barrier()

  def feats_src(k):
    row0 = pl.multiple_of(row_base + k * SUB, 8)
    return feats_hbm.at[pl.ds(row0, SUB), pl.ds(col0, DHALF)]

  def start_load(k):
    slot = lax.rem(k, NRING)
    return pltpu.async_copy(feats_src(k), fbuf.at[slot], lsem)

  start_load(0)

  def accum(k, _):
    slot = lax.rem(k, NRING)
    pltpu.make_async_copy(feats_src(k), fbuf.at[slot], lsem).wait()

    @pl.when(k + 1 < N_CHUNK)
    def _prefetch():
      start_load(k + 1)

    pltpu.sync_copy(fbuf.at[slot], sums_sh.at[ibuf.at[k]], add=True)
    return _
  lax.fori_loop(0, N_CHUNK, accum, 0)

  plsc.subcore_barrier()

  n_chunks = jnp.where(tile == NUM_SUBCORES - 1, LAST_SEG_CHUNKS,
                       FULL_SEG_CHUNKS)

  def finalize(k, _):
    base = pl.multiple_of(seg_base + k * SEG_CHUNK, 8)
    pltpu.sync_copy(sums_sh.at[pl.ds(base, SEG_CHUNK)], sbuf)
    pltpu.sync_copy(cnt_hbm.at[pl.ds(base, SEG_CHUNK)], cbuf)

    def div_one(s, _2):
      cntv = cbuf[s, :]
      inv = (ones16 / jnp.maximum(cntv, ones16))[0]
      for v in range(DHALF // LANES):
        sl = pl.ds(v * LANES, LANES)
        sbuf[s, sl] = sbuf[s, sl] * inv
      return _2
    lax.fori_loop(0, SEG_CHUNK, div_one, 0)

    pltpu.sync_copy(sbuf, out_hbm.at[pl.ds(base, SEG_CHUNK),
                                     pl.ds(col0, DHALF)])
    return _
  lax.fori_loop(0, n_chunks, finalize, 0)


@jax.jit
def _pool_mean(feats, batch3d):
  mesh = plsc.VectorSubcoreMesh(core_axis_name="c", subcore_axis_name="s")
  counts = pl.kernel(
      _counts_body,
      out_type=jax.ShapeDtypeStruct((S_PAD, LANES), jnp.float32),
      mesh=mesh,
      compiler_params=pltpu.CompilerParams(use_tc_tiling_on_sc=False),
      scratch_types=[
          pltpu.VMEM((N_CHUNK, SUB), jnp.int32),
          pltpu.VMEM((SUB, LANES), jnp.float32),
          pltpu.VMEM((SUB, LANES), jnp.float32),
          pltpu.VMEM_SHARED((S_PAD, LANES), jnp.float32),
          pltpu.SemaphoreType.DMA,
      ],
  )(batch3d)
  return pl.kernel(
      _pool_body,
      out_type=jax.ShapeDtypeStruct((N_SEG, N_FEATS), jnp.float32),
      mesh=mesh,
      scratch_types=[
          pltpu.VMEM((NRING, SUB, DHALF), jnp.float32),
          pltpu.VMEM((N_CHUNK, SUB), jnp.int32),
          pltpu.VMEM((SEG_CHUNK, DHALF), jnp.float32),
          pltpu.VMEM((SEG_CHUNK, LANES), jnp.float32),
          pltpu.VMEM_SHARED((S_PAD, DHALF), jnp.float32),
          pltpu.SemaphoreType.DMA,
      ],
  )(feats, batch3d, counts)


def kernel(feats, batch):
  batch3d = batch.astype(jnp.int32).reshape(NUM_SUBCORES,
                                            N_CHUNK, SUB)
  return _pool_mean(feats, batch3d)
